# baseline (speedup 1.0000x reference)
import functools

import jax
import jax.numpy as jnp
from jax import lax
from jax.experimental import pallas as pl
from jax.experimental.pallas import tpu as pltpu
from jax.experimental.pallas import tpu_sc as plsc

EMBED_DIM = 64
NUM_WORKERS = 32
HALF = 128


def _build_lookup(batch: int, hist: int, interpret: bool = False):
    rows_per_w = batch // NUM_WORKERS
    tail = hist - HALF
    mesh = plsc.VectorSubcoreMesh(core_axis_name="c", subcore_axis_name="s",
                                  num_cores=2, num_subcores=16)

    @functools.partial(
        pl.kernel,
        mesh=mesh,
        out_type=jax.ShapeDtypeStruct((batch, hist, EMBED_DIM), jnp.float32),
        scratch_types=[
            pltpu.VMEM((rows_per_w, hist), jnp.int32),
            pltpu.VMEM((HALF,), jnp.int32),
            pltpu.VMEM((32,), jnp.int32),
            pltpu.VMEM((HALF, 2 * EMBED_DIM), jnp.float32),
            pltpu.VMEM((HALF, 2 * EMBED_DIM), jnp.float32),
            pltpu.VMEM((HALF, EMBED_DIM), jnp.float32),
            pltpu.VMEM((tail, EMBED_DIM), jnp.float32),
            pltpu.SemaphoreType.DMA,
            pltpu.SemaphoreType.DMA,
            pltpu.SemaphoreType.DMA,
            pltpu.SemaphoreType.DMA,
        ],
        compiler_params=pltpu.CompilerParams(use_tc_tiling_on_sc=True,
                                             disable_bounds_checks=True,
                                             needs_layout_passes=False),
        interpret=interpret,
    )
    def lookup_kernel(seq_hbm, pairs_hbm, out_hbm, idx_v, pidx, offb,
                      rows0, rows1, o0, o1, sg0, sg1, so0, so1):
        wid = lax.axis_index("s") * 2 + lax.axis_index("c")
        base = wid * rows_per_w
        pltpu.sync_copy(seq_hbm.at[pl.ds(base, rows_per_w)], idx_v)

        zeros = jnp.zeros((16,), jnp.int32)

        def build_pidx(r, half):
            t0 = half * HALF
            ntok = HALF if half == 0 else tail
            starts = list(range(t0, t0 + ntok - 15, 16))
            if ntok % 16:
                starts.append(t0 + ntok - 16)
            for s in starts:
                v = idx_v[r, pl.ds(s, 16)]
                pidx[pl.ds(s - t0, 16)] = lax.shift_right_logical(v, 1)
            s = ntok
            while s + 16 <= HALF:
                pidx[pl.ds(s, 16)] = zeros
                s += 16
            if s < HALF:
                pidx[pl.ds(HALF - 16, 16)] = zeros

        def gather_desc(rows, sem):
            return pltpu.make_async_copy(pairs_hbm.at[pidx], rows, sem)

        def out_desc(r, half, obuf, sem):
            ncol = HALF if half == 0 else tail
            return pltpu.make_async_copy(
                obuf, out_hbm.at[base + r, pl.ds(half * HALF, ncol)], sem)

        lanes16 = lax.iota(jnp.int32, 16)

        def select(rows, obuf, r, half):
            t0 = half * HALF
            ntok = HALF if half == 0 else tail

            def do_block(s, toks):
                voff = (idx_v[r, pl.ds(t0 + s, 16)] & 1) * EMBED_DIM
                offb[pl.ds(0, 16)] = voff
                offb[pl.ds(16, 16)] = voff
                svec = jnp.broadcast_to(s, (16,))
                for t in toks:
                    osplat = plsc.load_gather(
                        offb, [jnp.full((16,), 16 + t, jnp.int32)])
                    rowv = svec + t
                    for k in range(0, EMBED_DIM, 16):
                        vals = plsc.load_gather(rows, [rowv, osplat + (k + lanes16)])
                        obuf[s + t, pl.ds(k, 16)] = vals

            def grp(g, carry):
                do_block(pl.multiple_of(16 * g, 16), range(16))
                return carry

            lax.fori_loop(0, ntok // 16, grp, 0)
            if ntok % 16:
                do_block(ntok - 16, range(16 - ntok % 16, 16))

        build_pidx(0, 0)
        gather_desc(rows0, sg0).start()

        def body(r, carry):
            for half, rows, obuf, sg, so in ((0, rows0, o0, sg0, so0),
                                             (1, rows1, o1, sg1, so1)):
                rows_o, sg_o = (rows1, sg1) if half == 0 else (rows0, sg0)
                obuf_o, so_o = (o1, so1) if half == 0 else (o0, so0)
                gather_desc(rows, sg).wait()
                if half == 0:
                    build_pidx(r, 1)
                    gather_desc(rows_o, sg_o).start()
                else:
                    @pl.when(r < rows_per_w - 1)
                    def _():
                        build_pidx(r + 1, 0)
                        gather_desc(rows_o, sg_o).start()
                select(rows, obuf, r, half)
                @pl.when((r >= 1) | (half == 1))
                def _():
                    pr = r if half == 1 else r - 1
                    out_desc(pr, 1 - half, obuf_o, so_o).wait()
                out_desc(r, half, obuf, so).start()
            return carry

        lax.fori_loop(0, rows_per_w, body, 0)
        out_desc(rows_per_w - 1, 1, o1, so1).wait()

    return lookup_kernel


def kernel(seq, node2vec):
    batch, hist = seq.shape
    vocab = node2vec.shape[0]
    pairs = node2vec.reshape(vocab // 2, 2 * EMBED_DIM)
    return _build_lookup(batch, hist)(seq.astype(jnp.int32), pairs)


# speedup vs baseline: 8.4414x; 8.4414x over previous
import functools

import jax
import jax.numpy as jnp
from jax import lax
from jax.experimental import pallas as pl
from jax.experimental.pallas import tpu as pltpu
from jax.experimental.pallas import tpu_sc as plsc

EMBED_DIM = 64
NUM_WORKERS = 32
CHUNK = 640


def _build_gather(total_rows: int):
    b_per_w = total_rows // NUM_WORKERS
    n_chunks = b_per_w // CHUNK
    assert n_chunks % 2 == 0
    mesh = plsc.VectorSubcoreMesh(core_axis_name="c", subcore_axis_name="s")

    @functools.partial(
        pl.kernel,
        mesh=mesh,
        out_type=jax.ShapeDtypeStruct((total_rows, EMBED_DIM), jnp.float32),
        scratch_types=[
            pltpu.VMEM((b_per_w,), jnp.int32),
            pltpu.VMEM((CHUNK, EMBED_DIM), jnp.float32),
            pltpu.VMEM((CHUNK, EMBED_DIM), jnp.float32),
            pltpu.SemaphoreType.DMA,
            pltpu.SemaphoreType.DMA,
            pltpu.SemaphoreType.DMA,
            pltpu.SemaphoreType.DMA,
        ],
        compiler_params=pltpu.CompilerParams(use_tc_tiling_on_sc=False),
    )
    def gather_kernel(idx_hbm, table_hbm, out_hbm, idx_v, rows0, rows1,
                      sg0, sg1, so0, so1):
        wid = lax.axis_index("s") * 2 + lax.axis_index("c")
        base = wid * b_per_w
        pltpu.sync_copy(idx_hbm.at[pl.ds(base, b_per_w)], idx_v)

        def gather_desc(i, rows, sem):
            return pltpu.make_async_copy(
                table_hbm.at[idx_v.at[pl.ds(i * CHUNK, CHUNK)]], rows, sem)

        def out_desc(i, rows, sem):
            return pltpu.make_async_copy(
                rows, out_hbm.at[pl.ds(base + i * CHUNK, CHUNK)], sem)

        gather_desc(0, rows0, sg0).start()

        def body(g, carry):
            for b, rows, sg, so in ((0, rows0, sg0, so0), (1, rows1, sg1, so1)):
                i = 2 * g + b
                rows_o, sg_o, so_o = (rows1, sg1, so1) if b == 0 else (rows0, sg0, so0)
                gather_desc(i, rows, sg).wait()
                out_desc(i, rows, so).start()
                @pl.when(i >= 1)
                def _():
                    out_desc(i - 1, rows_o, so_o).wait()
                @pl.when(i < n_chunks - 1)
                def _():
                    gather_desc(i + 1, rows_o, sg_o).start()
            return carry

        lax.fori_loop(0, n_chunks // 2, body, 0)
        out_desc(n_chunks - 1, rows1, so1).wait()

    return gather_kernel


def kernel(seq, node2vec):
    batch, hist = seq.shape
    flat_idx = seq.reshape(-1).astype(jnp.int32)
    out = _build_gather(flat_idx.shape[0])(flat_idx, node2vec)
    return out.reshape(batch, hist, EMBED_DIM)
